# baseline (speedup 1.0000x reference)
import functools

import jax
import jax.numpy as jnp
from jax import lax
from jax.experimental import pallas as pl
from jax.experimental.pallas import tpu as pltpu
from jax.experimental.pallas import tpu_sc as plsc

NC = 2
NS = 16
NW = NC * NS
K = 128
DDEG = 8
CH0_FRAC = 0.2


def _sc_mesh():
    return plsc.VectorSubcoreMesh(core_axis_name="c", subcore_axis_name="s")


_SC_PARAMS = pltpu.CompilerParams(use_tc_tiling_on_sc=False)


@functools.lru_cache(maxsize=None)
def _make_deg_kernel(ch0: int, cht: int, npad: int):
    rpt = npad // NS
    chb = max(ch0, cht - ch0)

    @functools.partial(
        pl.kernel,
        out_type=jax.ShapeDtypeStruct((NC, npad, DDEG), jnp.float32),
        mesh=_sc_mesh(),
        compiler_params=_SC_PARAMS,
        scratch_types=[
            pltpu.VMEM((chb, K), jnp.int32),
            pltpu.VMEM((K, DDEG), jnp.float32),
            pltpu.VMEM((rpt, DDEG), jnp.float32),
            pltpu.VMEM_SHARED((npad, DDEG), jnp.float32),
        ],
    )
    def deg_kernel(dst_hbm, ones_hbm, zeros_hbm, out_hbm,
                   dst_v, ones_v, stage_v, acc):
        c = lax.axis_index("c")
        s = lax.axis_index("s")
        cnt = jnp.where(c == 0, ch0, cht - ch0)

        @pl.when(c == 0)
        def _():
            pltpu.sync_copy(dst_hbm.at[s, pl.ds(0, ch0)],
                            dst_v.at[pl.ds(0, ch0)])

        @pl.when(c == 1)
        def _():
            pltpu.sync_copy(dst_hbm.at[s, pl.ds(ch0, cht - ch0)],
                            dst_v.at[pl.ds(0, cht - ch0)])

        pltpu.sync_copy(zeros_hbm, stage_v)
        pltpu.sync_copy(stage_v, acc.at[pl.ds(s * rpt, rpt)])
        pltpu.sync_copy(ones_hbm, ones_v)
        plsc.subcore_barrier()

        @pl.loop(0, cnt)
        def _(j):
            pltpu.sync_copy(ones_v, acc.at[dst_v.at[j]], add=True)

        plsc.subcore_barrier()
        pltpu.sync_copy(acc.at[pl.ds(s * rpt, rpt)], stage_v)
        pltpu.sync_copy(stage_v, out_hbm.at[c, pl.ds(s * rpt, rpt)])

    return deg_kernel


@functools.lru_cache(maxsize=None)
def _make_agg_kernel(ch0: int, cht: int, n: int, d: int, npad: int):
    rpt = npad // NS
    hpt = rpt // 2
    chb = max(ch0, cht - ch0)
    R = 4

    @functools.partial(
        pl.kernel,
        out_type=jax.ShapeDtypeStruct((NC, npad, d), jnp.float32),
        mesh=_sc_mesh(),
        compiler_params=_SC_PARAMS,
        scratch_types=[
            pltpu.VMEM((chb, K), jnp.int32),
            pltpu.VMEM((chb, K), jnp.int32),
            pltpu.VMEM((R, K, d), jnp.float32),
            pltpu.VMEM((hpt, d), jnp.float32),
            pltpu.VMEM_SHARED((npad, d), jnp.float32),
            pltpu.SemaphoreType.DMA,
        ],
    )
    def agg_kernel(rows_hbm, src_hbm, dst_hbm, zeros_hbm, out_hbm,
                   src_v, dst_v, rows_v, stage_v, acc, gsem):
        c = lax.axis_index("c")
        s = lax.axis_index("s")
        cnt = jnp.where(c == 0, ch0, cht - ch0)

        @pl.when(c == 0)
        def _():
            pltpu.sync_copy(src_hbm.at[s, pl.ds(0, ch0)],
                            src_v.at[pl.ds(0, ch0)])
            pltpu.sync_copy(dst_hbm.at[s, pl.ds(0, ch0)],
                            dst_v.at[pl.ds(0, ch0)])

        @pl.when(c == 1)
        def _():
            pltpu.sync_copy(src_hbm.at[s, pl.ds(ch0, cht - ch0)],
                            src_v.at[pl.ds(0, cht - ch0)])
            pltpu.sync_copy(dst_hbm.at[s, pl.ds(ch0, cht - ch0)],
                            dst_v.at[pl.ds(0, cht - ch0)])

        for b in range(R - 1):
            pltpu.async_copy(rows_hbm.at[src_v.at[b]], rows_v.at[b], gsem)
        pltpu.sync_copy(zeros_hbm, stage_v)
        for h in range(2):
            pltpu.sync_copy(stage_v,
                            acc.at[pl.ds(s * rpt + h * hpt, hpt)])
        plsc.subcore_barrier()

        @pl.loop(0, cnt, step=R)
        def _(j0):
            for b in range(R):
                j = j0 + b
                pltpu.make_async_copy(
                    rows_hbm.at[src_v.at[j]], rows_v.at[b], gsem).wait()

                @pl.when(j + R - 1 < cnt)
                def _issue():
                    pltpu.async_copy(rows_hbm.at[src_v.at[j + R - 1]],
                                     rows_v.at[(b + R - 1) % R], gsem)

                pltpu.sync_copy(rows_v.at[b], acc.at[dst_v.at[j]], add=True)

        plsc.subcore_barrier()
        for h in range(2):
            pltpu.sync_copy(acc.at[pl.ds(s * rpt + h * hpt, hpt)], stage_v)
            pltpu.sync_copy(stage_v,
                            out_hbm.at[c, pl.ds(s * rpt + h * hpt, hpt)])

    return agg_kernel


def _dense1_body(n, x_ref, w_ref, b_ref, ms_ref, w1_ref, degp_ref,
                 xs_ref, dinv_ref):
    x = x_ref[...]
    mean = jnp.mean(x, axis=0, keepdims=True)
    cen = x - mean * ms_ref[...]
    var = jnp.mean(cen * cen, axis=0, keepdims=True)
    h0 = w_ref[...] * cen / jnp.sqrt(var + 1e-5) + b_ref[...]
    xw = jnp.dot(h0, w1_ref[...], preferred_element_type=jnp.float32)
    deg = degp_ref[0, :n, 0:1] + degp_ref[1, :n, 0:1] + 1.0
    dinv = lax.rsqrt(deg)
    dinv_ref[...] = dinv
    xs_ref[...] = dinv * xw


def _dense2_body(n, accp_ref, xs_ref, dinv_ref, b1_ref, ys_ref):
    dinv = dinv_ref[...]
    t = dinv * (accp_ref[0, :n] + accp_ref[1, :n] + xs_ref[...]) + b1_ref[...]
    h = jnp.where(t >= 0, t, 0.1 * t)
    ys_ref[...] = dinv * h


def _dense3_body(n, accp_ref, ys_ref, dinv_ref, wmu_ref, bmu_ref,
                 wls_ref, bls_ref, mu_ref, ls_ref):
    base = dinv_ref[...] * (accp_ref[0, :n] + accp_ref[1, :n] + ys_ref[...])
    mu_ref[...] = jnp.dot(base, wmu_ref[...],
                          preferred_element_type=jnp.float32) + bmu_ref[...]
    ls_ref[...] = jnp.dot(base, wls_ref[...],
                          preferred_element_type=jnp.float32) + bls_ref[...]


def kernel(x, edge_index, gn_weight, gn_bias, gn_mean_scale,
           W1, b1, Wmu, bmu, Wls, bls):
    n, din = x.shape
    dh = W1.shape[1]
    dout = Wmu.shape[1]
    e = edge_index.shape[1]

    cht = -(-e // (NS * K))
    cht = -(-cht // 8) * 8
    ch0 = max(4, int(round(cht * CH0_FRAC / 4.0)) * 4)
    e_pad = NS * cht * K
    npad = -(-(n + 1) // (NS * 8)) * (NS * 8)

    src = jnp.pad(edge_index[0], (0, e_pad - e))
    dst = jnp.pad(edge_index[1], (0, e_pad - e), constant_values=n)
    src3 = src.reshape(NS, cht, K)
    dst3 = dst.reshape(NS, cht, K)

    ones_k = jnp.ones((K, DDEG), jnp.float32)
    zer1 = jnp.zeros((npad // NS, DDEG), jnp.float32)
    zer2 = jnp.zeros((npad // NS // 2, dh), jnp.float32)

    degp = _make_deg_kernel(ch0, cht, npad)(dst3, ones_k, zer1)

    xs, dinv = pl.pallas_call(
        functools.partial(_dense1_body, n),
        out_shape=[
            jax.ShapeDtypeStruct((n, dh), jnp.float32),
            jax.ShapeDtypeStruct((n, 1), jnp.float32),
        ],
    )(x, gn_weight.reshape(1, din), gn_bias.reshape(1, din),
      gn_mean_scale.reshape(1, din), W1, degp)

    agg = _make_agg_kernel(ch0, cht, n, dh, npad)

    acc1 = agg(xs, src3, dst3, zer2)

    ys = pl.pallas_call(
        functools.partial(_dense2_body, n),
        out_shape=jax.ShapeDtypeStruct((n, dh), jnp.float32),
    )(acc1, xs, dinv, b1.reshape(1, dh))

    acc2 = agg(ys, src3, dst3, zer2)

    mu, logstd = pl.pallas_call(
        functools.partial(_dense3_body, n),
        out_shape=[
            jax.ShapeDtypeStruct((n, dout), jnp.float32),
            jax.ShapeDtypeStruct((n, dout), jnp.float32),
        ],
    )(acc2, ys, dinv, Wmu, bmu.reshape(1, dout), Wls, bls.reshape(1, dout))

    return (mu, mu, logstd)


# speedup vs baseline: 1.1763x; 1.1763x over previous
import functools

import jax
import jax.numpy as jnp
from jax import lax
from jax.experimental import pallas as pl
from jax.experimental.pallas import tpu as pltpu
from jax.experimental.pallas import tpu_sc as plsc

NC = 2
NS = 16
NW = NC * NS
K = 128
DDEG = 8
CH0_FRAC = 0.8


def _sc_mesh():
    return plsc.VectorSubcoreMesh(core_axis_name="c", subcore_axis_name="s")


_SC_PARAMS = pltpu.CompilerParams(use_tc_tiling_on_sc=False)


@functools.lru_cache(maxsize=None)
def _make_deg_kernel(ch0: int, cht: int, npad: int):
    rpt = npad // NS
    chb = max(ch0, cht - ch0)

    @functools.partial(
        pl.kernel,
        out_type=jax.ShapeDtypeStruct((NC, npad, DDEG), jnp.float32),
        mesh=_sc_mesh(),
        compiler_params=_SC_PARAMS,
        scratch_types=[
            pltpu.VMEM((chb, K), jnp.int32),
            pltpu.VMEM((K, DDEG), jnp.float32),
            pltpu.VMEM((rpt, DDEG), jnp.float32),
            pltpu.VMEM_SHARED((npad, DDEG), jnp.float32),
        ],
    )
    def deg_kernel(dst_hbm, ones_hbm, zeros_hbm, out_hbm,
                   dst_v, ones_v, stage_v, acc):
        c = lax.axis_index("c")
        s = lax.axis_index("s")
        cnt = jnp.where(c == 0, ch0, cht - ch0)

        @pl.when(c == 0)
        def _():
            pltpu.sync_copy(dst_hbm.at[s, pl.ds(0, ch0)],
                            dst_v.at[pl.ds(0, ch0)])

        @pl.when(c == 1)
        def _():
            pltpu.sync_copy(dst_hbm.at[s, pl.ds(ch0, cht - ch0)],
                            dst_v.at[pl.ds(0, cht - ch0)])

        pltpu.sync_copy(zeros_hbm, stage_v)
        pltpu.sync_copy(stage_v, acc.at[pl.ds(s * rpt, rpt)])
        pltpu.sync_copy(ones_hbm, ones_v)
        plsc.subcore_barrier()

        @pl.loop(0, cnt)
        def _(j):
            pltpu.sync_copy(ones_v, acc.at[dst_v.at[j]], add=True)

        plsc.subcore_barrier()
        pltpu.sync_copy(acc.at[pl.ds(s * rpt, rpt)], stage_v)
        pltpu.sync_copy(stage_v, out_hbm.at[c, pl.ds(s * rpt, rpt)])

    return deg_kernel


@functools.lru_cache(maxsize=None)
def _make_agg_kernel(ch0: int, cht: int, n: int, d: int, npad: int):
    rpt = npad // NS
    hpt = rpt // 2
    chb = max(ch0, cht - ch0)
    R = 4

    @functools.partial(
        pl.kernel,
        out_type=jax.ShapeDtypeStruct((NC, npad, d), jnp.float32),
        mesh=_sc_mesh(),
        compiler_params=_SC_PARAMS,
        scratch_types=[
            pltpu.VMEM((chb, K), jnp.int32),
            pltpu.VMEM((chb, K), jnp.int32),
            pltpu.VMEM((R, K, d), jnp.float32),
            pltpu.VMEM((hpt, d), jnp.float32),
            pltpu.VMEM_SHARED((npad, d), jnp.float32),
            pltpu.SemaphoreType.DMA,
        ],
    )
    def agg_kernel(rows_hbm, src_hbm, dst_hbm, zeros_hbm, out_hbm,
                   src_v, dst_v, rows_v, stage_v, acc, gsem):
        c = lax.axis_index("c")
        s = lax.axis_index("s")
        cnt = jnp.where(c == 0, ch0, cht - ch0)

        @pl.when(c == 0)
        def _():
            pltpu.sync_copy(src_hbm.at[s, pl.ds(0, ch0)],
                            src_v.at[pl.ds(0, ch0)])
            pltpu.sync_copy(dst_hbm.at[s, pl.ds(0, ch0)],
                            dst_v.at[pl.ds(0, ch0)])

        @pl.when(c == 1)
        def _():
            pltpu.sync_copy(src_hbm.at[s, pl.ds(ch0, cht - ch0)],
                            src_v.at[pl.ds(0, cht - ch0)])
            pltpu.sync_copy(dst_hbm.at[s, pl.ds(ch0, cht - ch0)],
                            dst_v.at[pl.ds(0, cht - ch0)])

        for b in range(R - 1):
            pltpu.async_copy(rows_hbm.at[src_v.at[b]], rows_v.at[b], gsem)
        pltpu.sync_copy(zeros_hbm, stage_v)
        for h in range(2):
            pltpu.sync_copy(stage_v,
                            acc.at[pl.ds(s * rpt + h * hpt, hpt)])
        plsc.subcore_barrier()

        @pl.loop(0, cnt, step=R)
        def _(j0):
            for b in range(R):
                j = j0 + b
                pltpu.make_async_copy(
                    rows_hbm.at[src_v.at[j]], rows_v.at[b], gsem).wait()

                @pl.when(j + R - 1 < cnt)
                def _issue():
                    pltpu.async_copy(rows_hbm.at[src_v.at[j + R - 1]],
                                     rows_v.at[(b + R - 1) % R], gsem)

                pltpu.sync_copy(rows_v.at[b], acc.at[dst_v.at[j]], add=True)

        plsc.subcore_barrier()
        for h in range(2):
            pltpu.sync_copy(acc.at[pl.ds(s * rpt + h * hpt, hpt)], stage_v)
            pltpu.sync_copy(stage_v,
                            out_hbm.at[c, pl.ds(s * rpt + h * hpt, hpt)])

    return agg_kernel


def _dense1_body(n, x_ref, w_ref, b_ref, ms_ref, w1_ref, degp_ref,
                 xs_ref, dinv_ref):
    x = x_ref[...]
    mean = jnp.mean(x, axis=0, keepdims=True)
    cen = x - mean * ms_ref[...]
    var = jnp.mean(cen * cen, axis=0, keepdims=True)
    h0 = w_ref[...] * cen / jnp.sqrt(var + 1e-5) + b_ref[...]
    xw = jnp.dot(h0, w1_ref[...], preferred_element_type=jnp.float32)
    deg = degp_ref[0, :n, 0:1] + degp_ref[1, :n, 0:1] + 1.0
    dinv = lax.rsqrt(deg)
    dinv_ref[...] = dinv
    xs_ref[...] = dinv * xw


def _dense2_body(n, accp_ref, xs_ref, dinv_ref, b1_ref, ys_ref):
    dinv = dinv_ref[...]
    t = dinv * (accp_ref[0, :n] + accp_ref[1, :n] + xs_ref[...]) + b1_ref[...]
    h = jnp.where(t >= 0, t, 0.1 * t)
    ys_ref[...] = dinv * h


def _dense3_body(n, accp_ref, ys_ref, dinv_ref, wmu_ref, bmu_ref,
                 wls_ref, bls_ref, mu_ref, ls_ref):
    base = dinv_ref[...] * (accp_ref[0, :n] + accp_ref[1, :n] + ys_ref[...])
    mu_ref[...] = jnp.dot(base, wmu_ref[...],
                          preferred_element_type=jnp.float32) + bmu_ref[...]
    ls_ref[...] = jnp.dot(base, wls_ref[...],
                          preferred_element_type=jnp.float32) + bls_ref[...]


def kernel(x, edge_index, gn_weight, gn_bias, gn_mean_scale,
           W1, b1, Wmu, bmu, Wls, bls):
    n, din = x.shape
    dh = W1.shape[1]
    dout = Wmu.shape[1]
    e = edge_index.shape[1]

    cht = -(-e // (NS * K))
    cht = -(-cht // 8) * 8
    ch0 = max(4, int(round(cht * CH0_FRAC / 4.0)) * 4)
    e_pad = NS * cht * K
    npad = -(-(n + 1) // (NS * 8)) * (NS * 8)

    src = jnp.pad(edge_index[0], (0, e_pad - e))
    dst = jnp.pad(edge_index[1], (0, e_pad - e), constant_values=n)
    src3 = src.reshape(NS, cht, K)
    dst3 = dst.reshape(NS, cht, K)

    ones_k = jnp.ones((K, DDEG), jnp.float32)
    zer1 = jnp.zeros((npad // NS, DDEG), jnp.float32)
    zer2 = jnp.zeros((npad // NS // 2, dh), jnp.float32)

    degp = _make_deg_kernel(ch0, cht, npad)(dst3, ones_k, zer1)

    xs, dinv = pl.pallas_call(
        functools.partial(_dense1_body, n),
        out_shape=[
            jax.ShapeDtypeStruct((n, dh), jnp.float32),
            jax.ShapeDtypeStruct((n, 1), jnp.float32),
        ],
    )(x, gn_weight.reshape(1, din), gn_bias.reshape(1, din),
      gn_mean_scale.reshape(1, din), W1, degp)

    agg = _make_agg_kernel(ch0, cht, n, dh, npad)

    acc1 = agg(xs, src3, dst3, zer2)

    ys = pl.pallas_call(
        functools.partial(_dense2_body, n),
        out_shape=jax.ShapeDtypeStruct((n, dh), jnp.float32),
    )(acc1, xs, dinv, b1.reshape(1, dh))

    acc2 = agg(ys, src3, dst3, zer2)

    mu, logstd = pl.pallas_call(
        functools.partial(_dense3_body, n),
        out_shape=[
            jax.ShapeDtypeStruct((n, dout), jnp.float32),
            jax.ShapeDtypeStruct((n, dout), jnp.float32),
        ],
    )(acc2, ys, dinv, Wmu, bmu.reshape(1, dout), Wls, bls.reshape(1, dout))

    return (mu, mu, logstd)


# speedup vs baseline: 1.1764x; 1.0001x over previous
import functools

import jax
import jax.numpy as jnp
from jax import lax
from jax.experimental import pallas as pl
from jax.experimental.pallas import tpu as pltpu
from jax.experimental.pallas import tpu_sc as plsc

NC = 2
NS = 16
NW = NC * NS
K = 128
DDEG = 8
CH0_FRAC = 0.8


def _sc_mesh():
    return plsc.VectorSubcoreMesh(core_axis_name="c", subcore_axis_name="s")


_SC_PARAMS = pltpu.CompilerParams(use_tc_tiling_on_sc=False)


@functools.lru_cache(maxsize=None)
def _make_deg_kernel(ch0: int, cht: int, npad: int):
    rpt = npad // NS
    chb = max(ch0, cht - ch0)

    @functools.partial(
        pl.kernel,
        out_type=jax.ShapeDtypeStruct((NC, npad, DDEG), jnp.float32),
        mesh=_sc_mesh(),
        compiler_params=_SC_PARAMS,
        scratch_types=[
            pltpu.VMEM((chb, K), jnp.int32),
            pltpu.VMEM((K, DDEG), jnp.float32),
            pltpu.VMEM((rpt, DDEG), jnp.float32),
            pltpu.VMEM_SHARED((npad, DDEG), jnp.float32),
        ],
    )
    def deg_kernel(dst_hbm, ones_hbm, zeros_hbm, out_hbm,
                   dst_v, ones_v, stage_v, acc):
        c = lax.axis_index("c")
        s = lax.axis_index("s")

        @pl.when(c == 0)
        def _():
            pltpu.sync_copy(dst_hbm.at[s, pl.ds(0, ch0)],
                            dst_v.at[pl.ds(0, ch0)])

        @pl.when(c == 1)
        def _():
            pltpu.sync_copy(dst_hbm.at[s, pl.ds(ch0, cht - ch0)],
                            dst_v.at[pl.ds(0, cht - ch0)])

        pltpu.sync_copy(zeros_hbm, stage_v)
        pltpu.sync_copy(stage_v, acc.at[pl.ds(s * rpt, rpt)])
        pltpu.sync_copy(ones_hbm, ones_v)
        plsc.subcore_barrier()

        for core, cnt in ((0, ch0), (1, cht - ch0)):
            @pl.when(c == core)
            def _(cnt=cnt):
                @pl.loop(0, cnt)
                def _(j):
                    pltpu.sync_copy(ones_v, acc.at[dst_v.at[j]], add=True)

        plsc.subcore_barrier()
        pltpu.sync_copy(acc.at[pl.ds(s * rpt, rpt)], stage_v)
        pltpu.sync_copy(stage_v, out_hbm.at[c, pl.ds(s * rpt, rpt)])

    return deg_kernel


@functools.lru_cache(maxsize=None)
def _make_agg_kernel(ch0: int, cht: int, n: int, d: int, npad: int):
    rpt = npad // NS
    hpt = rpt // 2
    chb = max(ch0, cht - ch0)
    R = 4

    @functools.partial(
        pl.kernel,
        out_type=jax.ShapeDtypeStruct((NC, npad, d), jnp.float32),
        mesh=_sc_mesh(),
        compiler_params=_SC_PARAMS,
        scratch_types=[
            pltpu.VMEM((chb, K), jnp.int32),
            pltpu.VMEM((chb, K), jnp.int32),
            pltpu.VMEM((R, K, d), jnp.float32),
            pltpu.VMEM((hpt, d), jnp.float32),
            pltpu.VMEM_SHARED((npad, d), jnp.float32),
            pltpu.SemaphoreType.DMA,
        ],
    )
    def agg_kernel(rows_hbm, src_hbm, dst_hbm, zeros_hbm, out_hbm,
                   src_v, dst_v, rows_v, stage_v, acc, gsem):
        c = lax.axis_index("c")
        s = lax.axis_index("s")

        @pl.when(c == 0)
        def _():
            pltpu.sync_copy(src_hbm.at[s, pl.ds(0, ch0)],
                            src_v.at[pl.ds(0, ch0)])
            pltpu.sync_copy(dst_hbm.at[s, pl.ds(0, ch0)],
                            dst_v.at[pl.ds(0, ch0)])

        @pl.when(c == 1)
        def _():
            pltpu.sync_copy(src_hbm.at[s, pl.ds(ch0, cht - ch0)],
                            src_v.at[pl.ds(0, cht - ch0)])
            pltpu.sync_copy(dst_hbm.at[s, pl.ds(ch0, cht - ch0)],
                            dst_v.at[pl.ds(0, cht - ch0)])

        for b in range(R - 1):
            pltpu.async_copy(rows_hbm.at[src_v.at[b]], rows_v.at[b], gsem)
        pltpu.sync_copy(zeros_hbm, stage_v)
        for h in range(2):
            pltpu.sync_copy(stage_v,
                            acc.at[pl.ds(s * rpt + h * hpt, hpt)])
        plsc.subcore_barrier()

        for core, cnt in ((0, ch0), (1, cht - ch0)):
            @pl.when(c == core)
            def _(cnt=cnt):
                @pl.loop(0, cnt, step=R)
                def _(j0):
                    for b in range(R):
                        j = j0 + b
                        pltpu.make_async_copy(
                            rows_hbm.at[src_v.at[j]], rows_v.at[b],
                            gsem).wait()

                        @pl.when(j + R - 1 < cnt)
                        def _issue():
                            pltpu.async_copy(
                                rows_hbm.at[src_v.at[j + R - 1]],
                                rows_v.at[(b + R - 1) % R], gsem)

                        pltpu.sync_copy(rows_v.at[b], acc.at[dst_v.at[j]],
                                        add=True)

        plsc.subcore_barrier()
        for h in range(2):
            pltpu.sync_copy(acc.at[pl.ds(s * rpt + h * hpt, hpt)], stage_v)
            pltpu.sync_copy(stage_v,
                            out_hbm.at[c, pl.ds(s * rpt + h * hpt, hpt)])

    return agg_kernel


def _dense1_body(n, x_ref, w_ref, b_ref, ms_ref, w1_ref, degp_ref,
                 xs_ref, dinv_ref):
    x = x_ref[...]
    mean = jnp.mean(x, axis=0, keepdims=True)
    cen = x - mean * ms_ref[...]
    var = jnp.mean(cen * cen, axis=0, keepdims=True)
    h0 = w_ref[...] * cen / jnp.sqrt(var + 1e-5) + b_ref[...]
    xw = jnp.dot(h0, w1_ref[...], preferred_element_type=jnp.float32)
    deg = degp_ref[0, :n, 0:1] + degp_ref[1, :n, 0:1] + 1.0
    dinv = lax.rsqrt(deg)
    dinv_ref[...] = dinv
    xs_ref[...] = dinv * xw


def _dense2_body(n, accp_ref, xs_ref, dinv_ref, b1_ref, ys_ref):
    dinv = dinv_ref[...]
    t = dinv * (accp_ref[0, :n] + accp_ref[1, :n] + xs_ref[...]) + b1_ref[...]
    h = jnp.where(t >= 0, t, 0.1 * t)
    ys_ref[...] = dinv * h


def _dense3_body(n, accp_ref, ys_ref, dinv_ref, wmu_ref, bmu_ref,
                 wls_ref, bls_ref, mu_ref, ls_ref):
    base = dinv_ref[...] * (accp_ref[0, :n] + accp_ref[1, :n] + ys_ref[...])
    mu_ref[...] = jnp.dot(base, wmu_ref[...],
                          preferred_element_type=jnp.float32) + bmu_ref[...]
    ls_ref[...] = jnp.dot(base, wls_ref[...],
                          preferred_element_type=jnp.float32) + bls_ref[...]


def kernel(x, edge_index, gn_weight, gn_bias, gn_mean_scale,
           W1, b1, Wmu, bmu, Wls, bls):
    n, din = x.shape
    dh = W1.shape[1]
    dout = Wmu.shape[1]
    e = edge_index.shape[1]

    cht = -(-e // (NS * K))
    cht = -(-cht // 8) * 8
    ch0 = max(4, int(round(cht * CH0_FRAC / 4.0)) * 4)
    e_pad = NS * cht * K
    npad = -(-(n + 1) // (NS * 8)) * (NS * 8)

    src = jnp.pad(edge_index[0], (0, e_pad - e))
    dst = jnp.pad(edge_index[1], (0, e_pad - e), constant_values=n)
    src3 = src.reshape(NS, cht, K)
    dst3 = dst.reshape(NS, cht, K)

    ones_k = jnp.ones((K, DDEG), jnp.float32)
    zer1 = jnp.zeros((npad // NS, DDEG), jnp.float32)
    zer2 = jnp.zeros((npad // NS // 2, dh), jnp.float32)

    degp = _make_deg_kernel(ch0, cht, npad)(dst3, ones_k, zer1)

    xs, dinv = pl.pallas_call(
        functools.partial(_dense1_body, n),
        out_shape=[
            jax.ShapeDtypeStruct((n, dh), jnp.float32),
            jax.ShapeDtypeStruct((n, 1), jnp.float32),
        ],
    )(x, gn_weight.reshape(1, din), gn_bias.reshape(1, din),
      gn_mean_scale.reshape(1, din), W1, degp)

    agg = _make_agg_kernel(ch0, cht, n, dh, npad)

    acc1 = agg(xs, src3, dst3, zer2)

    ys = pl.pallas_call(
        functools.partial(_dense2_body, n),
        out_shape=jax.ShapeDtypeStruct((n, dh), jnp.float32),
    )(acc1, xs, dinv, b1.reshape(1, dh))

    acc2 = agg(ys, src3, dst3, zer2)

    mu, logstd = pl.pallas_call(
        functools.partial(_dense3_body, n),
        out_shape=[
            jax.ShapeDtypeStruct((n, dout), jnp.float32),
            jax.ShapeDtypeStruct((n, dout), jnp.float32),
        ],
    )(acc2, ys, dinv, Wmu, bmu.reshape(1, dout), Wls, bls.reshape(1, dout))

    return (mu, mu, logstd)
